# baseline (speedup 1.0000x reference)
import functools

import jax
import jax.numpy as jnp
from jax import lax
from jax.experimental import pallas as pl
from jax.experimental.pallas import tpu as pltpu
from jax.experimental.pallas import tpu_sc as plsc

_NC = 2
_NS = 16
_NW = _NC * _NS
_L = 16
_CHUNK = 128


@functools.lru_cache(maxsize=None)
def _build(n_nodes: int, dim: int, e_total: int):
    assert dim % (2 * _L) == 0
    assert e_total % _CHUNK == 0
    n_chunks = e_total // _CHUNK
    iters = (n_chunks + _NW - 1) // _NW
    n_sub = dim // _L
    n_pk = dim // (2 * _L)
    row_blk = next(b for b in range(min(_CHUNK, n_nodes), 0, -1)
                   if b % 8 == 0 and n_nodes % b == 0)
    n_row_blocks = n_nodes // row_blk
    blk_iters = (n_row_blocks + _NS - 1) // _NS

    mesh = plsc.VectorSubcoreMesh(core_axis_name="c", subcore_axis_name="s")

    @functools.partial(
        pl.kernel,
        mesh=mesh,
        out_type=jax.ShapeDtypeStruct((e_total,), jnp.float32),
        compiler_params=pltpu.CompilerParams(needs_layout_passes=False,
                                             use_tc_tiling_on_sc=False),
        scratch_types=[
            pltpu.HBM((_NC, n_nodes, dim // 2), jnp.int32),
            pltpu.HBM((_NC, n_nodes, dim // 2), jnp.int32),
            pltpu.VMEM((dim,), jnp.float32),
            pltpu.VMEM((row_blk, dim), jnp.float32),
            pltpu.VMEM((row_blk, dim // 2), jnp.int32),
            pltpu.VMEM((row_blk, dim // 2), jnp.int32),
            [pltpu.VMEM((_CHUNK,), jnp.int32) for _ in range(2)],
            [pltpu.VMEM((_CHUNK,), jnp.int32) for _ in range(2)],
            [pltpu.VMEM((_CHUNK, dim // 2), jnp.int32) for _ in range(2)],
            [pltpu.VMEM((_CHUNK, dim // 2), jnp.int32) for _ in range(2)],
            [pltpu.VMEM((_CHUNK,), jnp.float32) for _ in range(2)],
            pltpu.VMEM((_L * 17,), jnp.float32),
            [pltpu.SemaphoreType.DMA for _ in range(10)],
        ],
    )
    def _sc_kernel(table, rel, src, dst, out, packed_s, packed_p, kvec_v,
                   fbuf, pbuf_s, pbuf_p, idx_s, idx_d, rows_a, rows_b, out_v,
                   tmat, sems):
        cid = lax.axis_index("c")
        sid = lax.axis_index("s")
        wid = sid * _NC + cid
        sem_is, sem_id, sem_a, sem_b, sem_o = (
            sems[0:2], sems[2:4], sems[4:6], sems[6:8], sems[8:10])
        pltpu.sync_copy(rel, kvec_v)
        k_regs = [kvec_v[pl.ds(_L * j, _L)] for j in range(n_sub)]
        lane = lax.iota(jnp.int32, _L)
        lane17 = lane * 17
        src_tab = packed_s.at[cid]
        dst_tab = packed_p.at[cid]

        for t in range(blk_iters):
            blk = t * _NS + sid

            @pl.when(blk < n_row_blocks)
            def _():
                r0 = blk * row_blk
                pltpu.sync_copy(table.at[pl.ds(r0, row_blk)], fbuf)

                def pack_row(r, carry):
                    f = [fbuf[r, pl.ds(_L * j, _L)] for j in range(n_sub)]
                    fk = [f[j] * k_regs[j] for j in range(n_sub)]
                    for pbuf, vals in ((pbuf_s, fk), (pbuf_p, f)):
                        for p in range(n_pk):
                            pk = plsc.pack(vals[2 * p], vals[2 * p + 1],
                                           format=plsc.PackFormat.INTERLEAVED)
                            pbuf[r, pl.ds(_L * p, _L)] = (
                                plsc.bitcast(pk, jnp.int32))
                    return carry

                lax.fori_loop(0, row_blk, pack_row, 0)
                pltpu.sync_copy(pbuf_s, src_tab.at[pl.ds(r0, row_blk)])
                pltpu.sync_copy(pbuf_p, dst_tab.at[pl.ds(r0, row_blk)])
        plsc.subcore_barrier()

        def start_idx(i, sl):
            c = i * _NW + wid

            @pl.when(c < n_chunks)
            def _():
                base = c * _CHUNK
                pltpu.async_copy(src.at[pl.ds(base, _CHUNK)], idx_s[sl],
                                 sem_is[sl])
                pltpu.async_copy(dst.at[pl.ds(base, _CHUNK)], idx_d[sl],
                                 sem_id[sl])

        def fire_rows(i, sl):
            c = i * _NW + wid

            @pl.when(c < n_chunks)
            def _():
                base = c * _CHUNK
                pltpu.make_async_copy(src.at[pl.ds(base, _CHUNK)], idx_s[sl],
                                      sem_is[sl]).wait()
                pltpu.make_async_copy(dst.at[pl.ds(base, _CHUNK)], idx_d[sl],
                                      sem_id[sl]).wait()
                pltpu.async_copy(src_tab.at[idx_s[sl]], rows_a[sl], sem_a[sl])
                pltpu.async_copy(dst_tab.at[idx_d[sl]], rows_b[sl], sem_b[sl])

        def finish(i, sl):
            c = i * _NW + wid

            @pl.when(c < n_chunks)
            def _():
                pltpu.make_async_copy(src_tab.at[idx_s[sl]], rows_a[sl],
                                      sem_a[sl]).wait()
                pltpu.make_async_copy(dst_tab.at[idx_d[sl]], rows_b[sl],
                                      sem_b[sl]).wait()
                @pl.when(c >= 2 * _NW)
                def _():
                    pltpu.make_async_copy(
                        out_v[sl],
                        out.at[pl.ds((c - 2 * _NW) * _CHUNK, _CHUNK)],
                        sem_o[sl]).wait()

                ra, rb = rows_a[sl], rows_b[sl]

                def _tree_sum(vs):
                    vs = list(vs)
                    while len(vs) > 1:
                        nxt = [vs[i] + vs[i + 1]
                               for i in range(0, len(vs) - 1, 2)]
                        if len(vs) % 2:
                            nxt.append(vs[-1])
                        vs = nxt
                    return vs[0]

                def group_body(g, gcarry):
                    sums = []
                    for t in range(_L):
                        e = g * _L + t
                        fs = []
                        for p in range(n_pk):
                            a = plsc.bitcast(ra[e, pl.ds(_L * p, _L)],
                                             jnp.bfloat16)
                            b = plsc.bitcast(rb[e, pl.ds(_L * p, _L)],
                                             jnp.bfloat16)
                            u0, u1 = plsc.unpack(
                                a * b, format=plsc.PackFormat.INTERLEAVED)
                            fs += [u0, u1]
                        sums.append(_tree_sum(fs))
                    base = g * _L
                    out_v[sl][pl.ds(base, _L)] = jnp.zeros((_L,), jnp.float32)
                    for t in range(_L):
                        plsc.addupdate_scatter(
                            out_v[sl], [jnp.full((_L,), base + t)], sums[t])
                    return gcarry

                lax.fori_loop(0, _CHUNK // _L, group_body, 0)
                pltpu.async_copy(out_v[sl], out.at[pl.ds(c * _CHUNK, _CHUNK)],
                                 sem_o[sl])

        start_idx(0, 0)
        fire_rows(0, 0)
        start_idx(1, 1)

        def body(ii, carry):
            for off in range(2):
                i = ii * 2 + off
                sl = off
                fire_rows(i + 1, 1 - sl)
                finish(i, sl)
                start_idx(i + 2, sl)
            return carry

        half = (iters + 1) // 2
        lax.fori_loop(0, half, body, 0)

        for sl in range(2):
            cand = [i for i in range(2 * half) if i % 2 == sl][-2:]
            i_lo, i_hi = cand
            c_lo, c_hi = i_lo * _NW + wid, i_hi * _NW + wid

            def _wait_out(c, sl=sl):
                pltpu.make_async_copy(out_v[sl],
                                      out.at[pl.ds(c * _CHUNK, _CHUNK)],
                                      sem_o[sl]).wait()

            @pl.when(c_hi < n_chunks)
            def _():
                _wait_out(c_hi)

            @pl.when(jnp.logical_and(c_hi >= n_chunks, c_lo < n_chunks))
            def _():
                _wait_out(c_lo)

    return _sc_kernel


def kernel(emb_node, rel_embedding, edge_pos_index, edge_neg_index):
    n_nodes, dim = emb_node.shape
    n_edges = edge_pos_index.shape[1]
    src = jnp.concatenate(
        [edge_pos_index[0], edge_neg_index[0]]).astype(jnp.int32)
    dst = jnp.concatenate(
        [edge_pos_index[1], edge_neg_index[1]]).astype(jnp.int32)
    rel = rel_embedding[0]
    scores = _build(n_nodes, dim, 2 * n_edges)(emb_node, rel, src, dst)
    return scores[:n_edges], scores[n_edges:]


# speedup vs baseline: 1.1103x; 1.1103x over previous
import functools

import jax
import jax.numpy as jnp
from jax import lax
from jax.experimental import pallas as pl
from jax.experimental.pallas import tpu as pltpu
from jax.experimental.pallas import tpu_sc as plsc

_NC = 2
_NS = 16
_NW = _NC * _NS
_L = 16
_CHUNK = 128


@functools.lru_cache(maxsize=None)
def _build(n_nodes: int, dim: int, e_total: int):
    assert dim % (2 * _L) == 0
    assert e_total % _CHUNK == 0
    n_chunks = e_total // _CHUNK
    iters = (n_chunks + _NW - 1) // _NW
    n_sub = dim // _L
    n_pk = dim // (2 * _L)
    row_blk = next(b for b in range(min(_CHUNK, n_nodes), 0, -1)
                   if b % 8 == 0 and n_nodes % b == 0)
    n_row_blocks = n_nodes // row_blk
    blk_iters = (n_row_blocks + _NS - 1) // _NS

    mesh = plsc.VectorSubcoreMesh(core_axis_name="c", subcore_axis_name="s")

    @functools.partial(
        pl.kernel,
        mesh=mesh,
        out_type=jax.ShapeDtypeStruct((e_total,), jnp.float32),
        compiler_params=pltpu.CompilerParams(needs_layout_passes=False,
                                             use_tc_tiling_on_sc=False),
        scratch_types=[
            pltpu.HBM((_NC, n_nodes, dim // 2), jnp.int32),
            pltpu.HBM((_NC, n_nodes, dim // 2), jnp.int32),
            pltpu.VMEM((dim,), jnp.float32),
            pltpu.VMEM((row_blk, dim), jnp.float32),
            pltpu.VMEM((row_blk, dim // 2), jnp.int32),
            pltpu.VMEM((row_blk, dim // 2), jnp.int32),
            [pltpu.VMEM((_CHUNK,), jnp.int32) for _ in range(2)],
            [pltpu.VMEM((_CHUNK,), jnp.int32) for _ in range(2)],
            [pltpu.VMEM((_CHUNK, dim // 2), jnp.int32) for _ in range(2)],
            [pltpu.VMEM((_CHUNK, dim // 2), jnp.int32) for _ in range(2)],
            [pltpu.VMEM((_CHUNK,), jnp.float32) for _ in range(2)],
            [pltpu.VMEM((_L * 17,), jnp.float32) for _ in range(2)],
            [pltpu.SemaphoreType.DMA for _ in range(10)],
        ],
    )
    def _sc_kernel(table, rel, src, dst, out, packed_s, packed_p, kvec_v,
                   fbuf, pbuf_s, pbuf_p, idx_s, idx_d, rows_a, rows_b, out_v,
                   tmat, sems):
        cid = lax.axis_index("c")
        sid = lax.axis_index("s")
        wid = sid * _NC + cid
        sem_is, sem_id, sem_a, sem_b, sem_o = (
            sems[0:2], sems[2:4], sems[4:6], sems[6:8], sems[8:10])
        pltpu.sync_copy(rel, kvec_v)
        k_regs = [kvec_v[pl.ds(_L * j, _L)] for j in range(n_sub)]
        lane = lax.iota(jnp.int32, _L)
        lane17 = lane * 17
        src_tab = packed_s.at[cid]
        dst_tab = packed_p.at[cid]

        for t in range(blk_iters):
            blk = t * _NS + sid

            @pl.when(blk < n_row_blocks)
            def _():
                r0 = blk * row_blk
                pltpu.sync_copy(table.at[pl.ds(r0, row_blk)], fbuf)

                def pack_row(r, carry):
                    f = [fbuf[r, pl.ds(_L * j, _L)] for j in range(n_sub)]
                    fk = [f[j] * k_regs[j] for j in range(n_sub)]
                    for pbuf, vals in ((pbuf_s, fk), (pbuf_p, f)):
                        for p in range(n_pk):
                            pk = plsc.pack(vals[2 * p], vals[2 * p + 1],
                                           format=plsc.PackFormat.INTERLEAVED)
                            pbuf[r, pl.ds(_L * p, _L)] = (
                                plsc.bitcast(pk, jnp.int32))
                    return carry

                lax.fori_loop(0, row_blk, pack_row, 0)
                pltpu.sync_copy(pbuf_s, src_tab.at[pl.ds(r0, row_blk)])
                pltpu.sync_copy(pbuf_p, dst_tab.at[pl.ds(r0, row_blk)])
        plsc.subcore_barrier()

        def start_idx(i, sl):
            c = i * _NW + wid

            @pl.when(c < n_chunks)
            def _():
                base = c * _CHUNK
                pltpu.async_copy(src.at[pl.ds(base, _CHUNK)], idx_s[sl],
                                 sem_is[sl])
                pltpu.async_copy(dst.at[pl.ds(base, _CHUNK)], idx_d[sl],
                                 sem_id[sl])

        def fire_rows(i, sl):
            c = i * _NW + wid

            @pl.when(c < n_chunks)
            def _():
                base = c * _CHUNK
                pltpu.make_async_copy(src.at[pl.ds(base, _CHUNK)], idx_s[sl],
                                      sem_is[sl]).wait()
                pltpu.make_async_copy(dst.at[pl.ds(base, _CHUNK)], idx_d[sl],
                                      sem_id[sl]).wait()
                pltpu.async_copy(src_tab.at[idx_s[sl]], rows_a[sl], sem_a[sl])
                pltpu.async_copy(dst_tab.at[idx_d[sl]], rows_b[sl], sem_b[sl])

        def finish(i, sl):
            c = i * _NW + wid

            @pl.when(c < n_chunks)
            def _():
                pltpu.make_async_copy(src_tab.at[idx_s[sl]], rows_a[sl],
                                      sem_a[sl]).wait()
                pltpu.make_async_copy(dst_tab.at[idx_d[sl]], rows_b[sl],
                                      sem_b[sl]).wait()
                @pl.when(c >= 2 * _NW)
                def _():
                    pltpu.make_async_copy(
                        out_v[sl],
                        out.at[pl.ds((c - 2 * _NW) * _CHUNK, _CHUNK)],
                        sem_o[sl]).wait()

                ra, rb = rows_a[sl], rows_b[sl]

                def _tree_sum(vs):
                    vs = list(vs)
                    while len(vs) > 1:
                        nxt = [vs[i] + vs[i + 1]
                               for i in range(0, len(vs) - 1, 2)]
                        if len(vs) % 2:
                            nxt.append(vs[-1])
                        vs = nxt
                    return vs[0]

                def emit_rows(g):
                    tm = tmat[g % 2]
                    res = _tree_sum([tm[pl.ds(r * 17, _L)]
                                     for r in range(_L)])
                    out_v[sl][pl.ds(g * _L, _L)] = res

                for g in range(_CHUNK // _L):
                    sums = []
                    for t in range(_L):
                        e = g * _L + t
                        fs = []
                        for p in range(n_pk):
                            a = plsc.bitcast(ra[e, pl.ds(_L * p, _L)],
                                             jnp.bfloat16)
                            b = plsc.bitcast(rb[e, pl.ds(_L * p, _L)],
                                             jnp.bfloat16)
                            u0, u1 = plsc.unpack(
                                a * b, format=plsc.PackFormat.INTERLEAVED)
                            fs += [u0, u1]
                        sums.append(_tree_sum(fs))
                    for t in range(_L):
                        plsc.store_scatter(tmat[g % 2], [lane17 + t], sums[t])
                    if g > 0:
                        emit_rows(g - 1)
                emit_rows(_CHUNK // _L - 1)
                pltpu.async_copy(out_v[sl], out.at[pl.ds(c * _CHUNK, _CHUNK)],
                                 sem_o[sl])

        start_idx(0, 0)
        fire_rows(0, 0)
        start_idx(1, 1)

        def body(ii, carry):
            for off in range(2):
                i = ii * 2 + off
                sl = off
                fire_rows(i + 1, 1 - sl)
                finish(i, sl)
                start_idx(i + 2, sl)
            return carry

        half = (iters + 1) // 2
        lax.fori_loop(0, half, body, 0)

        for sl in range(2):
            cand = [i for i in range(2 * half) if i % 2 == sl][-2:]
            i_lo, i_hi = cand
            c_lo, c_hi = i_lo * _NW + wid, i_hi * _NW + wid

            def _wait_out(c, sl=sl):
                pltpu.make_async_copy(out_v[sl],
                                      out.at[pl.ds(c * _CHUNK, _CHUNK)],
                                      sem_o[sl]).wait()

            @pl.when(c_hi < n_chunks)
            def _():
                _wait_out(c_hi)

            @pl.when(jnp.logical_and(c_hi >= n_chunks, c_lo < n_chunks))
            def _():
                _wait_out(c_lo)

    return _sc_kernel


def kernel(emb_node, rel_embedding, edge_pos_index, edge_neg_index):
    n_nodes, dim = emb_node.shape
    n_edges = edge_pos_index.shape[1]
    src = jnp.concatenate(
        [edge_pos_index[0], edge_neg_index[0]]).astype(jnp.int32)
    dst = jnp.concatenate(
        [edge_pos_index[1], edge_neg_index[1]]).astype(jnp.int32)
    rel = rel_embedding[0]
    scores = _build(n_nodes, dim, 2 * n_edges)(emb_node, rel, src, dst)
    return scores[:n_edges], scores[n_edges:]


# speedup vs baseline: 1.9567x; 1.7623x over previous
import functools

import jax
import jax.numpy as jnp
from jax import lax
from jax.experimental import pallas as pl
from jax.experimental.pallas import tpu as pltpu
from jax.experimental.pallas import tpu_sc as plsc

_NC = 2
_NS = 16
_NW = _NC * _NS
_L = 16
_CHUNK = 128


@functools.lru_cache(maxsize=None)
def _build(n_nodes: int, dim: int, e_total: int):
    assert dim % (2 * _L) == 0
    assert e_total % _CHUNK == 0
    n_chunks = e_total // _CHUNK
    iters = (n_chunks + _NW - 1) // _NW
    n_sub = dim // _L
    n_pk = dim // (2 * _L)
    row_blk = next(b for b in range(min(_CHUNK, n_nodes), 0, -1)
                   if b % 8 == 0 and n_nodes % b == 0)
    n_row_blocks = n_nodes // row_blk
    blk_iters = (n_row_blocks + _NS - 1) // _NS

    mesh = plsc.VectorSubcoreMesh(core_axis_name="c", subcore_axis_name="s")

    @functools.partial(
        pl.kernel,
        mesh=mesh,
        out_type=jax.ShapeDtypeStruct((e_total,), jnp.float32),
        compiler_params=pltpu.CompilerParams(needs_layout_passes=False,
                                             use_tc_tiling_on_sc=False),
        scratch_types=[
            pltpu.HBM((_NC, n_nodes, dim // 2), jnp.int32),
            pltpu.HBM((_NC, n_nodes, dim // 2), jnp.int32),
            pltpu.VMEM((dim,), jnp.float32),
            pltpu.VMEM((row_blk, dim), jnp.float32),
            pltpu.VMEM((row_blk, dim // 2), jnp.int32),
            pltpu.VMEM((row_blk, dim // 2), jnp.int32),
            [pltpu.VMEM((_CHUNK,), jnp.int32) for _ in range(2)],
            [pltpu.VMEM((_CHUNK,), jnp.int32) for _ in range(2)],
            [pltpu.VMEM((_CHUNK, dim // 2), jnp.int32) for _ in range(2)],
            [pltpu.VMEM((_CHUNK, dim // 2), jnp.int32) for _ in range(2)],
            [pltpu.VMEM((_CHUNK,), jnp.float32) for _ in range(2)],
            [pltpu.VMEM((_L * 17,), jnp.float32) for _ in range(2)],
            [pltpu.SemaphoreType.DMA for _ in range(10)],
        ],
    )
    def _sc_kernel(table, rel, src, dst, out, packed_s, packed_p, kvec_v,
                   fbuf, pbuf_s, pbuf_p, idx_s, idx_d, rows_a, rows_b, out_v,
                   tmat, sems):
        cid = lax.axis_index("c")
        sid = lax.axis_index("s")
        wid = sid * _NC + cid
        sem_is, sem_id, sem_a, sem_b, sem_o = (
            sems[0:2], sems[2:4], sems[4:6], sems[6:8], sems[8:10])
        pltpu.sync_copy(rel, kvec_v)
        k_regs = [kvec_v[pl.ds(_L * j, _L)] for j in range(n_sub)]
        lane = lax.iota(jnp.int32, _L)
        lane17 = lane * 17
        src_tab = packed_s.at[cid]
        dst_tab = packed_p.at[cid]

        for t in range(blk_iters):
            blk = t * _NS + sid

            @pl.when(blk < n_row_blocks)
            def _():
                r0 = blk * row_blk
                pltpu.sync_copy(table.at[pl.ds(r0, row_blk)], fbuf)

                def pack_row(r, carry):
                    f = [fbuf[r, pl.ds(_L * j, _L)] for j in range(n_sub)]
                    fk = [f[j] * k_regs[j] for j in range(n_sub)]
                    for pbuf, vals in ((pbuf_s, fk), (pbuf_p, f)):
                        for p in range(n_pk):
                            pk = plsc.pack(vals[2 * p], vals[2 * p + 1],
                                           format=plsc.PackFormat.INTERLEAVED)
                            pbuf[r, pl.ds(_L * p, _L)] = (
                                plsc.bitcast(pk, jnp.int32))
                    return carry

                lax.fori_loop(0, row_blk, pack_row, 0)
                pltpu.sync_copy(pbuf_s, src_tab.at[pl.ds(r0, row_blk)])
                pltpu.sync_copy(pbuf_p, dst_tab.at[pl.ds(r0, row_blk)])
        plsc.subcore_barrier()

        def start_idx(i, sl):
            c = i * _NW + wid

            @pl.when(c < n_chunks)
            def _():
                base = c * _CHUNK
                pltpu.async_copy(src.at[pl.ds(base, _CHUNK)], idx_s[sl],
                                 sem_is[sl])
                pltpu.async_copy(dst.at[pl.ds(base, _CHUNK)], idx_d[sl],
                                 sem_id[sl])

        def fire_rows(i, sl):
            c = i * _NW + wid

            @pl.when(c < n_chunks)
            def _():
                base = c * _CHUNK
                pltpu.make_async_copy(src.at[pl.ds(base, _CHUNK)], idx_s[sl],
                                      sem_is[sl]).wait()
                pltpu.make_async_copy(dst.at[pl.ds(base, _CHUNK)], idx_d[sl],
                                      sem_id[sl]).wait()
                pltpu.async_copy(src_tab.at[idx_s[sl]], rows_a[sl], sem_a[sl])
                pltpu.async_copy(dst_tab.at[idx_d[sl]], rows_b[sl], sem_b[sl])

        def finish(i, sl):
            c = i * _NW + wid

            @pl.when(c < n_chunks)
            def _():
                pltpu.make_async_copy(src_tab.at[idx_s[sl]], rows_a[sl],
                                      sem_a[sl]).wait()
                pltpu.make_async_copy(dst_tab.at[idx_d[sl]], rows_b[sl],
                                      sem_b[sl]).wait()
                @pl.when(c >= 2 * _NW)
                def _():
                    pltpu.make_async_copy(
                        out_v[sl],
                        out.at[pl.ds((c - 2 * _NW) * _CHUNK, _CHUNK)],
                        sem_o[sl]).wait()

                ra, rb = rows_a[sl], rows_b[sl]

                def _tree_sum(vs):
                    vs = list(vs)
                    while len(vs) > 1:
                        nxt = [vs[i] + vs[i + 1]
                               for i in range(0, len(vs) - 1, 2)]
                        if len(vs) % 2:
                            nxt.append(vs[-1])
                        vs = nxt
                    return vs[0]

                def calc_group(g, tm):
                    sums = []
                    for t in range(_L):
                        e = g * _L + t
                        fs = []
                        for p in range(n_pk):
                            a = plsc.bitcast(ra[e, pl.ds(_L * p, _L)],
                                             jnp.bfloat16)
                            b = plsc.bitcast(rb[e, pl.ds(_L * p, _L)],
                                             jnp.bfloat16)
                            u0, u1 = plsc.unpack(
                                a * b, format=plsc.PackFormat.INTERLEAVED)
                            fs += [u0, u1]
                        sums.append(_tree_sum(fs))
                    for t in range(_L):
                        plsc.store_scatter(tm, [lane17 + t], sums[t])

                def emit_rows(g, tm):
                    res = _tree_sum([tm[pl.ds(r * 17, _L)]
                                     for r in range(_L)])
                    out_v[sl][pl.ds(g * _L, _L)] = res

                def group_body(gg, gcarry):
                    g0 = gg * 2
                    calc_group(g0, tmat[0])
                    calc_group(g0 + 1, tmat[1])
                    emit_rows(g0, tmat[0])
                    emit_rows(g0 + 1, tmat[1])
                    return gcarry

                lax.fori_loop(0, _CHUNK // (2 * _L), group_body, 0)
                pltpu.async_copy(out_v[sl], out.at[pl.ds(c * _CHUNK, _CHUNK)],
                                 sem_o[sl])

        start_idx(0, 0)
        fire_rows(0, 0)
        start_idx(1, 1)

        def body(ii, carry):
            for off in range(2):
                i = ii * 2 + off
                sl = off
                fire_rows(i + 1, 1 - sl)
                finish(i, sl)
                start_idx(i + 2, sl)
            return carry

        half = (iters + 1) // 2
        lax.fori_loop(0, half, body, 0)

        for sl in range(2):
            cand = [i for i in range(2 * half) if i % 2 == sl][-2:]
            i_lo, i_hi = cand
            c_lo, c_hi = i_lo * _NW + wid, i_hi * _NW + wid

            def _wait_out(c, sl=sl):
                pltpu.make_async_copy(out_v[sl],
                                      out.at[pl.ds(c * _CHUNK, _CHUNK)],
                                      sem_o[sl]).wait()

            @pl.when(c_hi < n_chunks)
            def _():
                _wait_out(c_hi)

            @pl.when(jnp.logical_and(c_hi >= n_chunks, c_lo < n_chunks))
            def _():
                _wait_out(c_lo)

    return _sc_kernel


def kernel(emb_node, rel_embedding, edge_pos_index, edge_neg_index):
    n_nodes, dim = emb_node.shape
    n_edges = edge_pos_index.shape[1]
    src = jnp.concatenate(
        [edge_pos_index[0], edge_neg_index[0]]).astype(jnp.int32)
    dst = jnp.concatenate(
        [edge_pos_index[1], edge_neg_index[1]]).astype(jnp.int32)
    rel = rel_embedding[0]
    scores = _build(n_nodes, dim, 2 * n_edges)(emb_node, rel, src, dst)
    return scores[:n_edges], scores[n_edges:]
